# baseline (speedup 1.0000x reference)
import functools

import jax
import jax.numpy as jnp
from jax import lax
from jax.experimental import pallas as pl
from jax.experimental.pallas import tpu as pltpu
from jax.experimental.pallas import tpu_sc as plsc

_DIM = 32
_T = 2048
_NROWS = 2 * _T - 1
_VROW = _T * _DIM // 128
_NSH = 32
_TABR = (_NROWS + 1) * _DIM // 128


def _sc_copy_kernel(tab_hbm, out_hbm, sem):
    info = plsc.get_sparse_core_info()
    nc = info.num_cores
    nw = nc * info.num_subcores
    rows_per_w = _T // nw

    wid = lax.axis_index("s") * nc + lax.axis_index("c")
    base = wid * rows_per_w

    def issue(r, carry):
        o = _T - 1 - base - r
        k = lax.rem(o, _NSH)
        q = pl.multiple_of((o - k) // 4, 8)
        pltpu.make_async_copy(
            tab_hbm.at[k, pl.ds(q, _VROW)],
            out_hbm.at[base + r],
            sem,
        ).start()
        return carry

    def drain(r, carry):
        pltpu.make_async_copy(
            tab_hbm.at[0, pl.ds(0, _VROW)],
            out_hbm.at[base + r],
            sem,
        ).wait()
        return carry

    lax.fori_loop(0, rows_per_w, issue, 0)
    lax.fori_loop(0, rows_per_w, drain, 0)


def kernel(relative_embedding, T):
    del T
    mesh = plsc.VectorSubcoreMesh(core_axis_name="c", subcore_axis_name="s")
    run = functools.partial(
        pl.kernel,
        mesh=mesh,
        out_type=jax.ShapeDtypeStruct((_T, _VROW, 128), jnp.float32),
        scratch_types=[pltpu.SemaphoreType.DMA],
        compiler_params=pltpu.CompilerParams(use_tc_tiling_on_sc=True),
    )(_sc_copy_kernel)
    emb_pad = jnp.pad(relative_embedding, ((0, _NSH), (0, 0)))
    tab = jnp.stack([emb_pad[k : k + _NROWS + 1] for k in range(_NSH)])
    tab = tab.reshape(_NSH, _TABR, 128)
    out = run(tab)
    return out.reshape(_T, _T, _DIM)


# speedup vs baseline: 14.6143x; 14.6143x over previous
import functools

import jax
import jax.numpy as jnp
from jax import lax
from jax.experimental import pallas as pl
from jax.experimental.pallas import tpu as pltpu
from jax.experimental.pallas import tpu_sc as plsc

_DIM = 32
_T = 2048
_NROWS = 2 * _T - 1
_VROW = _T * _DIM // 128
_NSH = 32
_WIN = 1016


def _sc_copy_kernel(tab_hbm, out_hbm, window_v, sem):
    info = plsc.get_sparse_core_info()
    ns = info.num_subcores
    rows_per_w = _T // (info.num_cores * ns)

    c = lax.axis_index("c")
    s = lax.axis_index("s")
    m = ns * c + s
    k = _NSH - 1 - m

    pltpu.sync_copy(tab_hbm.at[k, pl.ds(0, _WIN)], window_v)

    def issue(t, carry):
        v = pl.multiple_of(_WIN - _VROW - 8 * t, 8)
        pltpu.make_async_copy(
            window_v.at[pl.ds(v, _VROW)],
            out_hbm.at[m + _NSH * t],
            sem,
        ).start()
        return carry

    def drain(t, carry):
        pltpu.make_async_copy(
            window_v.at[pl.ds(0, _VROW)],
            out_hbm.at[m + _NSH * t],
            sem,
        ).wait()
        return carry

    lax.fori_loop(0, rows_per_w, issue, 0)
    lax.fori_loop(0, rows_per_w, drain, 0)


def kernel(relative_embedding, T):
    del T
    mesh = plsc.VectorSubcoreMesh(core_axis_name="c", subcore_axis_name="s")
    run = functools.partial(
        pl.kernel,
        mesh=mesh,
        out_type=jax.ShapeDtypeStruct((_T, _VROW, 128), jnp.float32),
        scratch_types=[
            pltpu.VMEM((_WIN, 128), jnp.float32),
            pltpu.SemaphoreType.DMA,
        ],
        compiler_params=pltpu.CompilerParams(use_tc_tiling_on_sc=True),
    )(_sc_copy_kernel)
    nwin = _WIN * 128 // _DIM
    tab = jnp.stack([relative_embedding[k : k + nwin] for k in range(_NSH)])
    tab = tab.reshape(_NSH, _WIN, 128)
    out = run(tab)
    return out.reshape(_T, _T, _DIM)
